# Initial kernel scaffold; baseline (speedup 1.0000x reference)
import functools

import jax
import jax.numpy as jnp
from jax import lax
from jax.experimental import pallas as pl
from jax.experimental.pallas import tpu as pltpu
from jax.experimental.pallas import tpu_sc as plsc

B, T, D_IN = 64, 1024, 192
K, D_EMB = 512, 32
M = B * T

BM1 = 2048
BM3 = 4096

_NC, _NS = 2, 16
_NW = _NC * _NS
_BPW = M // _NW
_CHUNK = 128


def _enc_argmin_body(x_ref, we_ref, be_ref, cb_ref, ze_ref, idx_ref):
    xb = x_ref[...]
    z = jnp.dot(xb, we_ref[...]) + be_ref[...]
    ze_ref[...] = z
    cb = cb_ref[...]
    mm = lax.dot_general(z, cb, (((1,), (1,)), ((), ())))
    zs = jnp.sum(z * z, axis=1, keepdims=True)
    csq = jnp.sum(cb * cb, axis=1)
    d = zs - 2.0 * mm + csq[None, :]
    mval = jnp.min(d, axis=1, keepdims=True)
    ii = lax.broadcasted_iota(jnp.int32, d.shape, 1)
    sel = jnp.where(d == mval, ii, d.shape[1])
    idx_ref[...] = jnp.min(sel, axis=1, keepdims=True)


def _dec_body(zq_ref, wd_ref, bd_ref, xr_ref):
    xr_ref[...] = jnp.dot(zq_ref[...], wd_ref[...]) + bd_ref[...]


def _sc_gather(codebook, idx_flat):
    mesh = plsc.VectorSubcoreMesh(core_axis_name="c", subcore_axis_name="s")

    @functools.partial(
        pl.kernel,
        mesh=mesh,
        out_type=jax.ShapeDtypeStruct((M, D_EMB), jnp.float32),
        scratch_types=[
            pltpu.VMEM((_BPW,), jnp.int32),
            pltpu.VMEM((_BPW, D_EMB), jnp.float32),
            pltpu.SemaphoreType.DMA,
        ],
    )
    def gather_kernel(cb_hbm, idx_hbm, out_hbm, idx_v, rows_v, sem):
        wid = lax.axis_index("s") * _NC + lax.axis_index("c")
        base = wid * _BPW
        pltpu.sync_copy(idx_hbm.at[pl.ds(base, _BPW)], idx_v)
        copies = [
            pltpu.async_copy(
                cb_hbm.at[idx_v.at[pl.ds(j * _CHUNK, _CHUNK)]],
                rows_v.at[pl.ds(j * _CHUNK, _CHUNK)],
                sem,
            )
            for j in range(_BPW // _CHUNK)
        ]
        for c in copies:
            c.wait()
        pltpu.sync_copy(rows_v, out_hbm.at[pl.ds(base, _BPW)])

    return gather_kernel(codebook, idx_flat)


def kernel(x, W_enc, b_enc, codebook, W_dec, b_dec):
    x2 = x.reshape(M, D_IN)

    z_e, idx = pl.pallas_call(
        _enc_argmin_body,
        grid=(M // BM1,),
        in_specs=[
            pl.BlockSpec((BM1, D_IN), lambda i: (i, 0)),
            pl.BlockSpec((D_IN, D_EMB), lambda i: (0, 0)),
            pl.BlockSpec((1, D_EMB), lambda i: (0, 0)),
            pl.BlockSpec((K, D_EMB), lambda i: (0, 0)),
        ],
        out_specs=[
            pl.BlockSpec((BM1, D_EMB), lambda i: (i, 0)),
            pl.BlockSpec((BM1, 1), lambda i: (i, 0)),
        ],
        out_shape=[
            jax.ShapeDtypeStruct((M, D_EMB), jnp.float32),
            jax.ShapeDtypeStruct((M, 1), jnp.int32),
        ],
    )(x2, W_enc, b_enc.reshape(1, D_EMB), codebook)

    z_q = _sc_gather(codebook, idx.reshape(M))

    x_rec = pl.pallas_call(
        _dec_body,
        grid=(M // BM3,),
        in_specs=[
            pl.BlockSpec((BM3, D_EMB), lambda i: (i, 0)),
            pl.BlockSpec((D_EMB, D_IN), lambda i: (0, 0)),
            pl.BlockSpec((1, D_IN), lambda i: (0, 0)),
        ],
        out_specs=pl.BlockSpec((BM3, D_IN), lambda i: (i, 0)),
        out_shape=jax.ShapeDtypeStruct((M, D_IN), jnp.float32),
    )(z_q, W_dec, b_dec.reshape(1, D_IN))

    return (
        x_rec.reshape(B, T, D_IN),
        z_e.reshape(B, T, D_EMB),
        z_q.reshape(B, T, D_EMB),
    )


# speedup vs baseline: 1.1322x; 1.1322x over previous
import functools

import jax
import jax.numpy as jnp
from jax import lax
from jax.experimental import pallas as pl
from jax.experimental.pallas import tpu as pltpu
from jax.experimental.pallas import tpu_sc as plsc

B, T, D_IN = 64, 1024, 192
K, D_EMB = 512, 32
M = B * T

BM1 = 2048
BM3 = 4096

_NC, _NS = 2, 16
_NW = _NC * _NS
_BPW = M // _NW
_CHUNK = 128


def _enc_argmin_body(x_ref, we_ref, be_ref, cb_ref, ze_ref, idx_ref):
    xb = x_ref[...]
    z = jnp.dot(xb, we_ref[...]) + be_ref[...]
    ze_ref[...] = z
    cb = cb_ref[...]
    mm = lax.dot_general(z, cb, (((1,), (1,)), ((), ())))
    zs = jnp.sum(z * z, axis=1, keepdims=True)
    csq = jnp.sum(cb * cb, axis=1)
    d = zs - 2.0 * mm + csq[None, :]
    mval = jnp.min(d, axis=1, keepdims=True)
    ii = lax.broadcasted_iota(jnp.int32, d.shape, 1)
    sel = jnp.where(d == mval, ii, d.shape[1])
    idx_ref[...] = jnp.min(sel, axis=1, keepdims=True)


def _dec_body(zq_ref, wd_ref, bd_ref, xr_ref):
    xr_ref[...] = jnp.dot(zq_ref[...], wd_ref[...]) + bd_ref[...]


def _sc_gather(codebook, idx_flat):
    mesh = plsc.VectorSubcoreMesh(core_axis_name="c", subcore_axis_name="s")

    @functools.partial(
        pl.kernel,
        mesh=mesh,
        out_type=jax.ShapeDtypeStruct((M, D_EMB), jnp.float32),
        compiler_params=pltpu.CompilerParams(use_tc_tiling_on_sc=False),
        scratch_types=[
            pltpu.VMEM((_BPW,), jnp.int32),
            pltpu.VMEM((_BPW, D_EMB), jnp.float32),
            pltpu.SemaphoreType.DMA,
        ],
    )
    def gather_kernel(cb_hbm, idx_hbm, out_hbm, idx_v, rows_v, sem):
        wid = lax.axis_index("s") * _NC + lax.axis_index("c")
        base = wid * _BPW
        pltpu.sync_copy(idx_hbm.at[pl.ds(base, _BPW)], idx_v)
        copies = [
            pltpu.async_copy(
                cb_hbm.at[idx_v.at[pl.ds(j * _CHUNK, _CHUNK)]],
                rows_v.at[pl.ds(j * _CHUNK, _CHUNK)],
                sem,
            )
            for j in range(_BPW // _CHUNK)
        ]
        for c in copies:
            c.wait()
        pltpu.sync_copy(rows_v, out_hbm.at[pl.ds(base, _BPW)])

    return gather_kernel(codebook, idx_flat)


def kernel(x, W_enc, b_enc, codebook, W_dec, b_dec):
    x2 = x.reshape(M, D_IN)

    z_e, idx = pl.pallas_call(
        _enc_argmin_body,
        grid=(M // BM1,),
        in_specs=[
            pl.BlockSpec((BM1, D_IN), lambda i: (i, 0)),
            pl.BlockSpec((D_IN, D_EMB), lambda i: (0, 0)),
            pl.BlockSpec((1, D_EMB), lambda i: (0, 0)),
            pl.BlockSpec((K, D_EMB), lambda i: (0, 0)),
        ],
        out_specs=[
            pl.BlockSpec((BM1, D_EMB), lambda i: (i, 0)),
            pl.BlockSpec((BM1, 1), lambda i: (i, 0)),
        ],
        out_shape=[
            jax.ShapeDtypeStruct((M, D_EMB), jnp.float32),
            jax.ShapeDtypeStruct((M, 1), jnp.int32),
        ],
    )(x2, W_enc, b_enc.reshape(1, D_EMB), codebook)

    z_q = _sc_gather(codebook, idx.reshape(M))

    x_rec = pl.pallas_call(
        _dec_body,
        grid=(M // BM3,),
        in_specs=[
            pl.BlockSpec((BM3, D_EMB), lambda i: (i, 0)),
            pl.BlockSpec((D_EMB, D_IN), lambda i: (0, 0)),
            pl.BlockSpec((1, D_IN), lambda i: (0, 0)),
        ],
        out_specs=pl.BlockSpec((BM3, D_IN), lambda i: (i, 0)),
        out_shape=jax.ShapeDtypeStruct((M, D_IN), jnp.float32),
    )(z_q, W_dec, b_dec.reshape(1, D_IN))

    return (
        x_rec.reshape(B, T, D_IN),
        z_e.reshape(B, T, D_EMB),
        z_q.reshape(B, T, D_EMB),
    )
